# baseline (speedup 1.0000x reference)
import jax
import jax.numpy as jnp
from jax.experimental import pallas as pl
from jax.experimental.pallas import tpu as pltpu

MAX_LEN = 2048
EMBED_DIM = 768


def _copy_body(table_ref, out_ref, sem):
    pltpu.make_async_copy(table_ref, out_ref, sem).start()
    pltpu.make_async_copy(table_ref, out_ref, sem).wait()


@jax.jit
def _tc_copy(table):
    return pl.pallas_call(
        _copy_body,
        in_specs=[pl.BlockSpec(memory_space=pl.ANY)],
        out_specs=pl.BlockSpec(memory_space=pl.ANY),
        scratch_shapes=[pltpu.SemaphoreType.DMA],
        out_shape=jax.ShapeDtypeStruct((MAX_LEN, EMBED_DIM), jnp.float32),
    )(table)


def kernel(x, table):
    del x
    return _tc_copy(table)[None]


# speedup vs baseline: 40.5515x; 40.5515x over previous
import jax
import jax.numpy as jnp
from jax.experimental import pallas as pl
from jax.experimental.pallas import tpu as pltpu

MAX_LEN = 2048
EMBED_DIM = 768
HALF = MAX_LEN // 2


def _copy_body(table_ref, out_ref, buf0, buf1, s0, s1, t0, t1):
    g0 = pltpu.make_async_copy(table_ref.at[pl.ds(0, HALF)], buf0, s0)
    g1 = pltpu.make_async_copy(table_ref.at[pl.ds(HALF, HALF)], buf1, s1)
    g0.start()
    g1.start()
    g0.wait()
    p0 = pltpu.make_async_copy(buf0, out_ref.at[pl.ds(0, HALF)], t0)
    p0.start()
    g1.wait()
    p1 = pltpu.make_async_copy(buf1, out_ref.at[pl.ds(HALF, HALF)], t1)
    p1.start()
    p0.wait()
    p1.wait()


@jax.jit
def _tc_copy(table):
    return pl.pallas_call(
        _copy_body,
        in_specs=[pl.BlockSpec(memory_space=pl.ANY)],
        out_specs=pl.BlockSpec(memory_space=pl.ANY),
        scratch_shapes=[
            pltpu.VMEM((HALF, EMBED_DIM), jnp.float32),
            pltpu.VMEM((HALF, EMBED_DIM), jnp.float32),
            pltpu.SemaphoreType.DMA,
            pltpu.SemaphoreType.DMA,
            pltpu.SemaphoreType.DMA,
            pltpu.SemaphoreType.DMA,
        ],
        out_shape=jax.ShapeDtypeStruct((MAX_LEN, EMBED_DIM), jnp.float32),
    )(table)


def kernel(x, table):
    del x
    return _tc_copy(table)[None]
